# baseline (speedup 1.0000x reference)
import functools
import math

import jax
import jax.numpy as jnp
from jax.experimental import pallas as pl

B, N, C, D = 128, 1000, 100, 128
H, QKV = 8, 16
LOGIT_CLIP = 10.0
BB = 8


def _body(keepA_ref, keepB_ref, node_ref, ce_ref, vcm_ref, cur_ref, depot_ref,
          Wq_ref, Wk_ref, Wv_ref, Wks_ref, Wqm_ref, Wkm_ref, Wvm_ref, Wom_ref,
          aug_ref, ge_ref, gid_ref, logp_ref):
    f32 = jnp.float32
    node = node_ref[...]
    keepA = keepA_ref[...]
    keepB = keepB_ref[...]
    um = jnp.sum(node * keepA, axis=1) / N
    ucm = jnp.sum(node * keepB, axis=1) / N

    cur = cur_ref[:, 0, :]
    depot = depot_ref[:, 0, :]

    Wq = Wq_ref[...]
    q1 = (jnp.dot(um, Wq[0:D, :], preferred_element_type=f32)
          + jnp.dot(cur, Wq[D:2 * D, :], preferred_element_type=f32)
          + jnp.dot(depot, Wq[2 * D:3 * D, :], preferred_element_type=f32))
    qh = jnp.dot(q1, Wqm_ref[...], preferred_element_type=f32)

    Wkf = jnp.dot(Wk_ref[...], Wkm_ref[...], preferred_element_type=f32)
    Wvf = jnp.dot(Wv_ref[...], Wvm_ref[...], preferred_element_type=f32)

    ce = ce_ref[...]
    kh = jax.lax.dot_general(ce, Wkf, (((2,), (0,)), ((), ())),
                             preferred_element_type=f32)
    vh = jax.lax.dot_general(ce, Wvf, (((2,), (0,)), ((), ())),
                             preferred_element_type=f32)

    d_ids = jax.lax.broadcasted_iota(jnp.int32, (H * QKV, H), 0)
    h_ids = jax.lax.broadcasted_iota(jnp.int32, (H * QKV, H), 1)
    S = (d_ids // QKV == h_ids).astype(f32)

    prod = kh * qh[:, None, :]
    sc = jax.lax.dot_general(prod, S, (((2,), (0,)), ((), ())),
                             preferred_element_type=f32) / math.sqrt(QKV)

    vcm = vcm_ref[...]
    unvis = 1.0 - vcm
    rest = jnp.sum(unvis, axis=1, keepdims=True) - unvis[:, 0:1, :]
    all_vis = (rest == 0.0).astype(f32)
    c_ids = jax.lax.broadcasted_iota(jnp.int32, (BB, C, 1), 1)
    vcm_eff = jnp.where(c_ids == 0, 1.0 - all_vis, vcm)

    sc = jnp.where(vcm_eff > 0.0, -1e9, sc)
    mx = jnp.max(sc, axis=1, keepdims=True)
    e = jnp.exp(sc - mx)
    attn = e / jnp.sum(e, axis=1, keepdims=True)

    S2 = (d_ids // QKV == h_ids).astype(f32).T
    attn_l = jax.lax.dot_general(attn, S2, (((2,), (0,)), ((), ())),
                                 preferred_element_type=f32)
    out = jnp.sum(attn_l * vh, axis=1)

    Wlog = jax.lax.dot_general(Wom_ref[...], Wks_ref[...],
                               (((1,), (1,)), ((), ())),
                               preferred_element_type=f32)
    g = jnp.dot(out, Wlog, preferred_element_type=f32)

    logit = jnp.sum(ce * g[:, None, :], axis=2) / math.sqrt(D)
    logit = jnp.tanh(logit) * LOGIT_CLIP
    vcm2 = vcm_eff[:, :, 0]
    logit = jnp.where(vcm2 > 0.0, -1e9, logit)

    mx2 = jnp.max(logit, axis=1, keepdims=True)
    lse = jnp.log(jnp.sum(jnp.exp(logit - mx2), axis=1, keepdims=True)) + mx2
    logp = logit - lse
    logp_ref[...] = logp

    mxv = jnp.max(logp, axis=1, keepdims=True)
    idc = jax.lax.broadcasted_iota(jnp.int32, (BB, C), 1)
    cand = jnp.where(logp == mxv, idc, C)
    gid = jnp.min(cand, axis=1, keepdims=True)
    gid_ref[...] = gid

    onehot = (idc == gid).astype(f32)
    ge = jnp.sum(ce * onehot[:, :, None], axis=1)
    ge_ref[...] = ge[:, None, :]

    aug = jnp.concatenate([ucm, cur, ge, depot], axis=-1)
    aug_ref[...] = aug[:, None, :]


@functools.partial(jax.jit, static_argnames=())
def _run(keepA, keepB, node_embeddings, cluster_embedding, vcm_t,
         current_embedding, depot_embedding, Wq, Wk, Wv, Wks,
         Wq_m, Wk_m, Wv_m, Wo_m):
    nb = B // BB
    f32 = jnp.float32
    bspec = pl.BlockSpec
    grid_spec = pl.GridSpec(
        grid=(nb,),
        in_specs=[
            bspec((BB, N, 1), lambda i: (i, 0, 0)),
            bspec((BB, N, 1), lambda i: (i, 0, 0)),
            bspec((BB, N, D), lambda i: (i, 0, 0)),
            bspec((BB, C, D), lambda i: (i, 0, 0)),
            bspec((BB, C, 1), lambda i: (i, 0, 0)),
            bspec((BB, 1, D), lambda i: (i, 0, 0)),
            bspec((BB, 1, D), lambda i: (i, 0, 0)),
            bspec((3 * D, D), lambda i: (0, 0)),
            bspec((D, D), lambda i: (0, 0)),
            bspec((D, D), lambda i: (0, 0)),
            bspec((D, D), lambda i: (0, 0)),
            bspec((D, H * QKV), lambda i: (0, 0)),
            bspec((D, H * QKV), lambda i: (0, 0)),
            bspec((D, H * QKV), lambda i: (0, 0)),
            bspec((H * QKV, D), lambda i: (0, 0)),
        ],
        out_specs=[
            bspec((BB, 1, 4 * D), lambda i: (i, 0, 0)),
            bspec((BB, 1, D), lambda i: (i, 0, 0)),
            bspec((BB, 1), lambda i: (i, 0)),
            bspec((BB, C), lambda i: (i, 0)),
        ],
    )
    out_shapes = [
        jax.ShapeDtypeStruct((B, 1, 4 * D), f32),
        jax.ShapeDtypeStruct((B, 1, D), f32),
        jax.ShapeDtypeStruct((B, 1), jnp.int32),
        jax.ShapeDtypeStruct((B, C), f32),
    ]
    return pl.pallas_call(_body, grid_spec=grid_spec, out_shape=out_shapes)(
        keepA, keepB, node_embeddings, cluster_embedding, vcm_t,
        current_embedding, depot_embedding, Wq, Wk, Wv, Wks,
        Wq_m, Wk_m, Wv_m, Wo_m)


def kernel(depot_embedding, cluster_embedding, current_embedding, node_embeddings,
           aug_context_embedding, is_new_cluster, cluster_mask, visited_cluster_mask,
           mask, cluster_guidance_embedding, select_mode, cluster_guidance, step,
           Wq, Wk, Wv, Wks, Wq_m, Wk_m, Wv_m, Wo_m):
    f32 = jnp.float32
    keepA = (~mask).astype(f32).transpose(0, 2, 1)
    keepB = (~(mask | cluster_mask)).astype(f32).transpose(0, 2, 1)
    vcm_t = visited_cluster_mask.astype(f32).transpose(0, 2, 1)
    aug, ge, gid, logp = _run(
        keepA, keepB, node_embeddings, cluster_embedding, vcm_t,
        current_embedding, depot_embedding, Wq, Wk, Wv, Wks,
        Wq_m, Wk_m, Wv_m, Wo_m)
    return (aug, ge, gid.reshape(B), logp)


# speedup vs baseline: 2.1178x; 2.1178x over previous
import functools
import math

import jax
import jax.numpy as jnp
from jax.experimental import pallas as pl

B, N, C, D = 128, 1000, 100, 128
H, QKV = 8, 16
LOGIT_CLIP = 10.0
BB = 8


def _body(keep2_ref, node_ref, ce_ref, vcm_ref, cur_ref, depot_ref,
          Wq_ref, Wk_ref, Wv_ref, Wks_ref, Wqm_ref, Wkm_ref, Wvm_ref, Wom_ref,
          aug_ref, ge_ref, gid_ref, logp_ref):
    f32 = jnp.float32
    node = node_ref[...]
    keep2 = keep2_ref[...]
    sums = jax.lax.dot_general(keep2, node, (((2,), (1,)), ((0,), (0,))),
                               preferred_element_type=f32)
    um = sums[:, 0, :] / N
    ucm = sums[:, 1, :] / N

    cur = cur_ref[:, 0, :]
    depot = depot_ref[:, 0, :]

    Wq = Wq_ref[...]
    q1 = (jnp.dot(um, Wq[0:D, :], preferred_element_type=f32)
          + jnp.dot(cur, Wq[D:2 * D, :], preferred_element_type=f32)
          + jnp.dot(depot, Wq[2 * D:3 * D, :], preferred_element_type=f32))
    qh = jnp.dot(q1, Wqm_ref[...], preferred_element_type=f32)

    Wkf = jnp.dot(Wk_ref[...], Wkm_ref[...], preferred_element_type=f32)
    Wvf = jnp.dot(Wv_ref[...], Wvm_ref[...], preferred_element_type=f32)

    ce = ce_ref[...]
    kh = jax.lax.dot_general(ce, Wkf, (((2,), (0,)), ((), ())),
                             preferred_element_type=f32)
    vh = jax.lax.dot_general(ce, Wvf, (((2,), (0,)), ((), ())),
                             preferred_element_type=f32)

    d_ids = jax.lax.broadcasted_iota(jnp.int32, (H * QKV, H), 0)
    h_ids = jax.lax.broadcasted_iota(jnp.int32, (H * QKV, H), 1)
    S = (d_ids // QKV == h_ids).astype(f32)

    prod = kh * qh[:, None, :]
    sc = jax.lax.dot_general(prod, S, (((2,), (0,)), ((), ())),
                             preferred_element_type=f32) / math.sqrt(QKV)

    vcm = vcm_ref[...]
    unvis = 1.0 - vcm
    rest = jnp.sum(unvis, axis=1, keepdims=True) - unvis[:, 0:1, :]
    all_vis = (rest == 0.0).astype(f32)
    c_ids = jax.lax.broadcasted_iota(jnp.int32, (BB, C, 1), 1)
    vcm_eff = jnp.where(c_ids == 0, 1.0 - all_vis, vcm)

    sc = jnp.where(vcm_eff > 0.0, -1e9, sc)
    mx = jnp.max(sc, axis=1, keepdims=True)
    e = jnp.exp(sc - mx)
    attn = e / jnp.sum(e, axis=1, keepdims=True)

    S2 = (d_ids // QKV == h_ids).astype(f32).T
    attn_l = jax.lax.dot_general(attn, S2, (((2,), (0,)), ((), ())),
                                 preferred_element_type=f32)
    out = jnp.sum(attn_l * vh, axis=1)

    Wlog = jax.lax.dot_general(Wom_ref[...], Wks_ref[...],
                               (((1,), (1,)), ((), ())),
                               preferred_element_type=f32)
    g = jnp.dot(out, Wlog, preferred_element_type=f32)

    logit = jnp.sum(ce * g[:, None, :], axis=2) / math.sqrt(D)
    logit = jnp.tanh(logit) * LOGIT_CLIP
    vcm2 = vcm_eff[:, :, 0]
    logit = jnp.where(vcm2 > 0.0, -1e9, logit)

    mx2 = jnp.max(logit, axis=1, keepdims=True)
    lse = jnp.log(jnp.sum(jnp.exp(logit - mx2), axis=1, keepdims=True)) + mx2
    logp = logit - lse
    logp_ref[...] = logp

    mxv = jnp.max(logp, axis=1, keepdims=True)
    idc = jax.lax.broadcasted_iota(jnp.int32, (BB, C), 1)
    cand = jnp.where(logp == mxv, idc, C)
    gid = jnp.min(cand, axis=1, keepdims=True)
    gid_ref[...] = gid

    onehot = (idc == gid).astype(f32)
    ge = jnp.sum(ce * onehot[:, :, None], axis=1)
    ge_ref[...] = ge[:, None, :]

    aug = jnp.concatenate([ucm, cur, ge, depot], axis=-1)
    aug_ref[...] = aug[:, None, :]


@functools.partial(jax.jit, static_argnames=())
def _run(keep2, node_embeddings, cluster_embedding, vcm_t,
         current_embedding, depot_embedding, Wq, Wk, Wv, Wks,
         Wq_m, Wk_m, Wv_m, Wo_m):
    nb = B // BB
    f32 = jnp.float32
    bspec = pl.BlockSpec
    grid_spec = pl.GridSpec(
        grid=(nb,),
        in_specs=[
            bspec((BB, 2, N), lambda i: (i, 0, 0)),
            bspec((BB, N, D), lambda i: (i, 0, 0)),
            bspec((BB, C, D), lambda i: (i, 0, 0)),
            bspec((BB, C, 1), lambda i: (i, 0, 0)),
            bspec((BB, 1, D), lambda i: (i, 0, 0)),
            bspec((BB, 1, D), lambda i: (i, 0, 0)),
            bspec((3 * D, D), lambda i: (0, 0)),
            bspec((D, D), lambda i: (0, 0)),
            bspec((D, D), lambda i: (0, 0)),
            bspec((D, D), lambda i: (0, 0)),
            bspec((D, H * QKV), lambda i: (0, 0)),
            bspec((D, H * QKV), lambda i: (0, 0)),
            bspec((D, H * QKV), lambda i: (0, 0)),
            bspec((H * QKV, D), lambda i: (0, 0)),
        ],
        out_specs=[
            bspec((BB, 1, 4 * D), lambda i: (i, 0, 0)),
            bspec((BB, 1, D), lambda i: (i, 0, 0)),
            bspec((BB, 1), lambda i: (i, 0)),
            bspec((BB, C), lambda i: (i, 0)),
        ],
    )
    out_shapes = [
        jax.ShapeDtypeStruct((B, 1, 4 * D), f32),
        jax.ShapeDtypeStruct((B, 1, D), f32),
        jax.ShapeDtypeStruct((B, 1), jnp.int32),
        jax.ShapeDtypeStruct((B, C), f32),
    ]
    return pl.pallas_call(_body, grid_spec=grid_spec, out_shape=out_shapes)(
        keep2, node_embeddings, cluster_embedding, vcm_t,
        current_embedding, depot_embedding, Wq, Wk, Wv, Wks,
        Wq_m, Wk_m, Wv_m, Wo_m)


def kernel(depot_embedding, cluster_embedding, current_embedding, node_embeddings,
           aug_context_embedding, is_new_cluster, cluster_mask, visited_cluster_mask,
           mask, cluster_guidance_embedding, select_mode, cluster_guidance, step,
           Wq, Wk, Wv, Wks, Wq_m, Wk_m, Wv_m, Wo_m):
    f32 = jnp.float32
    keepA = (~mask).astype(f32)
    keepB = (~(mask | cluster_mask)).astype(f32)
    keep2 = jnp.concatenate([keepA, keepB], axis=1)
    vcm_t = visited_cluster_mask.astype(f32).transpose(0, 2, 1)
    aug, ge, gid, logp = _run(
        keep2, node_embeddings, cluster_embedding, vcm_t,
        current_embedding, depot_embedding, Wq, Wk, Wv, Wks,
        Wq_m, Wk_m, Wv_m, Wo_m)
    return (aug, ge, gid.reshape(B), logp)


# speedup vs baseline: 2.1181x; 1.0002x over previous
import functools
import math

import jax
import jax.numpy as jnp
from jax.experimental import pallas as pl
from jax.experimental.pallas import tpu as pltpu

B, N, C, D = 128, 1000, 100, 128
H, QKV = 8, 16
LOGIT_CLIP = 10.0
BB = 8


def _body(keep2_ref, node_ref, ce_ref, vcm_ref, cur_ref, depot_ref,
          Wq_ref, Wk_ref, Wv_ref, Wks_ref, Wqm_ref, Wkm_ref, Wvm_ref, Wom_ref,
          aug_ref, ge_ref, gid_ref, logp_ref):
    f32 = jnp.float32
    node = node_ref[...]
    keep2 = keep2_ref[...]
    sums = jax.lax.dot_general(keep2, node, (((2,), (1,)), ((0,), (0,))),
                               preferred_element_type=f32)
    um = sums[:, 0, :] / N
    ucm = sums[:, 1, :] / N

    cur = cur_ref[:, 0, :]
    depot = depot_ref[:, 0, :]

    Wq = Wq_ref[...]
    q1 = (jnp.dot(um, Wq[0:D, :], preferred_element_type=f32)
          + jnp.dot(cur, Wq[D:2 * D, :], preferred_element_type=f32)
          + jnp.dot(depot, Wq[2 * D:3 * D, :], preferred_element_type=f32))
    qh = jnp.dot(q1, Wqm_ref[...], preferred_element_type=f32)

    Wkf = jnp.dot(Wk_ref[...], Wkm_ref[...], preferred_element_type=f32)
    Wvf = jnp.dot(Wv_ref[...], Wvm_ref[...], preferred_element_type=f32)

    ce = ce_ref[...]
    kh = jax.lax.dot_general(ce, Wkf, (((2,), (0,)), ((), ())),
                             preferred_element_type=f32)
    vh = jax.lax.dot_general(ce, Wvf, (((2,), (0,)), ((), ())),
                             preferred_element_type=f32)

    d_ids = jax.lax.broadcasted_iota(jnp.int32, (H * QKV, H), 0)
    h_ids = jax.lax.broadcasted_iota(jnp.int32, (H * QKV, H), 1)
    S = (d_ids // QKV == h_ids).astype(f32)

    prod = kh * qh[:, None, :]
    sc = jax.lax.dot_general(prod, S, (((2,), (0,)), ((), ())),
                             preferred_element_type=f32) / math.sqrt(QKV)

    vcm = vcm_ref[...]
    unvis = 1.0 - vcm
    rest = jnp.sum(unvis, axis=1, keepdims=True) - unvis[:, 0:1, :]
    all_vis = (rest == 0.0).astype(f32)
    c_ids = jax.lax.broadcasted_iota(jnp.int32, (BB, C, 1), 1)
    vcm_eff = jnp.where(c_ids == 0, 1.0 - all_vis, vcm)

    sc = jnp.where(vcm_eff > 0.0, -1e9, sc)
    mx = jnp.max(sc, axis=1, keepdims=True)
    e = jnp.exp(sc - mx)
    attn = e / jnp.sum(e, axis=1, keepdims=True)

    S2 = (d_ids // QKV == h_ids).astype(f32).T
    attn_l = jax.lax.dot_general(attn, S2, (((2,), (0,)), ((), ())),
                                 preferred_element_type=f32)
    out = jnp.sum(attn_l * vh, axis=1)

    Wlog = jax.lax.dot_general(Wom_ref[...], Wks_ref[...],
                               (((1,), (1,)), ((), ())),
                               preferred_element_type=f32)
    g = jnp.dot(out, Wlog, preferred_element_type=f32)

    logit = jnp.sum(ce * g[:, None, :], axis=2) / math.sqrt(D)
    logit = jnp.tanh(logit) * LOGIT_CLIP
    vcm2 = vcm_eff[:, :, 0]
    logit = jnp.where(vcm2 > 0.0, -1e9, logit)

    mx2 = jnp.max(logit, axis=1, keepdims=True)
    lse = jnp.log(jnp.sum(jnp.exp(logit - mx2), axis=1, keepdims=True)) + mx2
    logp = logit - lse
    logp_ref[...] = logp

    mxv = jnp.max(logp, axis=1, keepdims=True)
    idc = jax.lax.broadcasted_iota(jnp.int32, (BB, C), 1)
    cand = jnp.where(logp == mxv, idc, C)
    gid = jnp.min(cand, axis=1, keepdims=True)
    gid_ref[...] = gid

    onehot = (idc == gid).astype(f32)
    ge = jnp.sum(ce * onehot[:, :, None], axis=1)
    ge_ref[...] = ge[:, None, :]

    aug = jnp.concatenate([ucm, cur, ge, depot], axis=-1)
    aug_ref[...] = aug[:, None, :]


@functools.partial(jax.jit, static_argnames=())
def _run(keep2, node_embeddings, cluster_embedding, vcm_t,
         current_embedding, depot_embedding, Wq, Wk, Wv, Wks,
         Wq_m, Wk_m, Wv_m, Wo_m):
    nb = B // BB
    f32 = jnp.float32
    bspec = pl.BlockSpec
    grid_spec = pl.GridSpec(
        grid=(nb,),
        in_specs=[
            bspec((BB, 2, N), lambda i: (i, 0, 0)),
            bspec((BB, N, D), lambda i: (i, 0, 0)),
            bspec((BB, C, D), lambda i: (i, 0, 0)),
            bspec((BB, C, 1), lambda i: (i, 0, 0)),
            bspec((BB, 1, D), lambda i: (i, 0, 0)),
            bspec((BB, 1, D), lambda i: (i, 0, 0)),
            bspec((3 * D, D), lambda i: (0, 0)),
            bspec((D, D), lambda i: (0, 0)),
            bspec((D, D), lambda i: (0, 0)),
            bspec((D, D), lambda i: (0, 0)),
            bspec((D, H * QKV), lambda i: (0, 0)),
            bspec((D, H * QKV), lambda i: (0, 0)),
            bspec((D, H * QKV), lambda i: (0, 0)),
            bspec((H * QKV, D), lambda i: (0, 0)),
        ],
        out_specs=[
            bspec((BB, 1, 4 * D), lambda i: (i, 0, 0)),
            bspec((BB, 1, D), lambda i: (i, 0, 0)),
            bspec((BB, 1), lambda i: (i, 0)),
            bspec((BB, C), lambda i: (i, 0)),
        ],
    )
    out_shapes = [
        jax.ShapeDtypeStruct((B, 1, 4 * D), f32),
        jax.ShapeDtypeStruct((B, 1, D), f32),
        jax.ShapeDtypeStruct((B, 1), jnp.int32),
        jax.ShapeDtypeStruct((B, C), f32),
    ]
    return pl.pallas_call(
        _body, grid_spec=grid_spec, out_shape=out_shapes,
        compiler_params=pltpu.CompilerParams(
            dimension_semantics=("parallel",)),
    )(
        keep2, node_embeddings, cluster_embedding, vcm_t,
        current_embedding, depot_embedding, Wq, Wk, Wv, Wks,
        Wq_m, Wk_m, Wv_m, Wo_m)


def kernel(depot_embedding, cluster_embedding, current_embedding, node_embeddings,
           aug_context_embedding, is_new_cluster, cluster_mask, visited_cluster_mask,
           mask, cluster_guidance_embedding, select_mode, cluster_guidance, step,
           Wq, Wk, Wv, Wks, Wq_m, Wk_m, Wv_m, Wo_m):
    f32 = jnp.float32
    keepA = (~mask).astype(f32)
    keepB = (~(mask | cluster_mask)).astype(f32)
    keep2 = jnp.concatenate([keepA, keepB], axis=1)
    vcm_t = visited_cluster_mask.astype(f32).transpose(0, 2, 1)
    aug, ge, gid, logp = _run(
        keep2, node_embeddings, cluster_embedding, vcm_t,
        current_embedding, depot_embedding, Wq, Wk, Wv, Wks,
        Wq_m, Wk_m, Wv_m, Wo_m)
    return (aug, ge, gid.reshape(B), logp)
